# Initial kernel scaffold; baseline (speedup 1.0000x reference)
import jax
import jax.numpy as jnp
from jax.experimental import pallas as pl


def kernel(xyz, params):
    raise NotImplementedError("write your pallas kernel here")


# speedup vs baseline: 36.6046x; 36.6046x over previous
import functools
from functools import partial

import jax
import jax.numpy as jnp
from jax import lax
from jax.experimental import pallas as pl
from jax.experimental.pallas import tpu as pltpu

F32 = jnp.float32


def _bi(dtype, shape, dim):
    return lax.broadcasted_iota(dtype, shape, dim)


def _fps_body(xyzT_ref, out_ref, *, npoint):
    Bb = xyzT_ref.shape[0]
    n = xyzT_ref.shape[2]
    x = xyzT_ref[:, 0, :]
    y = xyzT_ref[:, 1, :]
    z = xyzT_ref[:, 2, :]
    iota_n = _bi(jnp.int32, (Bb, n), 1)
    iota_p = _bi(jnp.int32, (Bb, npoint), 1)

    def body(i, st):
        dists, far, ox, oy, oz = st
        oh = (iota_n == far).astype(F32)
        cx = jnp.sum(x * oh, axis=1, keepdims=True)
        cy = jnp.sum(y * oh, axis=1, keepdims=True)
        cz = jnp.sum(z * oh, axis=1, keepdims=True)
        d = (x - cx) ** 2 + (y - cy) ** 2 + (z - cz) ** 2
        dists = jnp.minimum(dists, d)
        m = jnp.max(dists, axis=1, keepdims=True)
        cand = jnp.where(dists == m, iota_n, n)
        far = jnp.min(cand, axis=1, keepdims=True)
        slot = iota_p == i
        ox = jnp.where(slot, cx, ox)
        oy = jnp.where(slot, cy, oy)
        oz = jnp.where(slot, cz, oz)
        return dists, far, ox, oy, oz

    dists0 = jnp.full((Bb, n), 1e10, F32)
    far0 = jnp.zeros((Bb, 1), jnp.int32)
    zp = jnp.zeros((Bb, npoint), F32)
    _, _, ox, oy, oz = lax.fori_loop(0, npoint, body, (dists0, far0, zp, zp, zp))
    out_ref[:, 0, :] = ox
    out_ref[:, 1, :] = oy
    out_ref[:, 2, :] = oz


def _fps(xyzT, npoint):
    Bb, _, n = xyzT.shape
    return pl.pallas_call(
        partial(_fps_body, npoint=npoint),
        out_shape=jax.ShapeDtypeStruct((Bb, 3, npoint), F32),
    )(xyzT)


def _sa_body(q_ref, srcT_ref, src_ref, wt_ref, b_ref, y_ref, st_ref, *,
             radius, ns, R, n_src, c_in):
    first = (pl.program_id(0) == 0) & (pl.program_id(1) == 0)

    @pl.when(first)
    def _():
        st_ref[...] = jnp.zeros_like(st_ref)

    q = q_ref[0]
    sx = srcT_ref[0, 0:1, :]
    sy = srcT_ref[0, 1:2, :]
    sz = srcT_ref[0, 2:3, :]
    qx, qy, qz = q[:, 0:1], q[:, 1:2], q[:, 2:3]
    qq = qx * qx + qy * qy + qz * qz
    ss = sx * sx + sy * sy + sz * sz
    cross = jnp.dot(q, srcT_ref[0], preferred_element_type=F32)
    d = (qq + ss) - 2.0 * cross
    mask = jnp.logical_not(d > radius * radius)
    maskf = mask.astype(F32)

    nch = n_src // 128
    tri128 = (_bi(jnp.int32, (128, 128), 0)
              <= _bi(jnp.int32, (128, 128), 1)).astype(F32)
    tric = (_bi(jnp.int32, (nch, nch), 0)
            < _bi(jnp.int32, (nch, nch), 1)).astype(F32)
    m3 = maskf.reshape(R, nch, 128)
    within = jnp.dot(m3.reshape(R * nch, 128), tri128,
                     preferred_element_type=F32).reshape(R, nch, 128)
    tot = jnp.sum(m3, axis=2)
    carry = jnp.dot(tot, tric, preferred_element_type=F32)
    pos = (within + carry[:, :, None]).reshape(R, n_src)

    jp1 = (_bi(jnp.int32, (1, ns, 1), 1) + 1).astype(F32)
    sel = ((pos[:, None, :] == jp1) & mask[:, None, :]).astype(F32)
    g = jnp.dot(sel.reshape(R * ns, n_src), src_ref[0],
                preferred_element_type=F32,
                precision=lax.Precision.HIGHEST).reshape(R, ns, c_in)

    cnt = jnp.sum(maskf, axis=1).reshape(R, 1, 1)
    g0 = g[:, 0:1, :]
    src0 = src_ref[0][0:1, :].reshape(1, 1, c_in)
    base = jnp.where(cnt > 0, g0, src0)
    jf = _bi(jnp.int32, (1, ns, 1), 1).astype(F32)
    g = jnp.where(jf < cnt, g, base)

    gx = g[:, :, 0:3] - q[:, None, :]
    if c_in > 3:
        g = jnp.concatenate([gx, g[:, :, 3:]], axis=2)
    else:
        g = gx

    x2 = g.reshape(R * ns, c_in)
    yv = jnp.dot(x2, wt_ref[...], preferred_element_type=F32) + b_ref[...]
    y_ref[0] = yv
    s1 = jnp.sum(yv, axis=0, keepdims=True)
    s2 = jnp.sum(yv * yv, axis=0, keepdims=True)
    st_ref[0:1, :] += s1
    st_ref[1:2, :] += s2


def _sa_group_conv(q, srcT, src, wt, bvec, radius, ns, R):
    Bb, S, _ = q.shape
    n_src = src.shape[1]
    c_in = src.shape[2]
    c_out = wt.shape[1]
    grid = (Bb, S // R)
    return pl.pallas_call(
        partial(_sa_body, radius=radius, ns=ns, R=R, n_src=n_src, c_in=c_in),
        grid=grid,
        in_specs=[
            pl.BlockSpec((1, R, 3), lambda b, s: (b, s, 0)),
            pl.BlockSpec((1, 3, n_src), lambda b, s: (b, 0, 0)),
            pl.BlockSpec((1, n_src, c_in), lambda b, s: (b, 0, 0)),
            pl.BlockSpec((c_in, c_out), lambda b, s: (0, 0)),
            pl.BlockSpec((1, c_out), lambda b, s: (0, 0)),
        ],
        out_specs=[
            pl.BlockSpec((1, R * ns, c_out), lambda b, s: (b, s, 0)),
            pl.BlockSpec((8, c_out), lambda b, s: (0, 0)),
        ],
        out_shape=[
            jax.ShapeDtypeStruct((Bb, S * ns, c_out), F32),
            jax.ShapeDtypeStruct((8, c_out), F32),
        ],
    )(q, srcT, src, wt, bvec)


def _mid_body(x_ref, st_ref, gam_ref, bet_ref, wt_ref, b_ref,
              y_ref, sto_ref, *, count):
    first = (pl.program_id(0) == 0) & (pl.program_id(1) == 0)

    @pl.when(first)
    def _():
        sto_ref[...] = jnp.zeros_like(sto_ref)

    st = st_ref[...]
    mu = st[0:1, :] / count
    var = st[1:2, :] / count - mu * mu
    inv = lax.rsqrt(var + 1e-5)
    x = x_ref[0]
    xh = jnp.maximum((x - mu) * inv * gam_ref[...] + bet_ref[...], 0.0)
    yv = jnp.dot(xh, wt_ref[...], preferred_element_type=F32) + b_ref[...]
    y_ref[0] = yv
    sto_ref[0:1, :] += jnp.sum(yv, axis=0, keepdims=True)
    sto_ref[1:2, :] += jnp.sum(yv * yv, axis=0, keepdims=True)


def _mid_layer(x, st, gam, bet, wt, bvec, count, Rp):
    Bb, P, c_in = x.shape
    c_out = wt.shape[1]
    grid = (Bb, P // Rp)
    return pl.pallas_call(
        partial(_mid_body, count=count),
        grid=grid,
        in_specs=[
            pl.BlockSpec((1, Rp, c_in), lambda b, s: (b, s, 0)),
            pl.BlockSpec((8, c_in), lambda b, s: (0, 0)),
            pl.BlockSpec((1, c_in), lambda b, s: (0, 0)),
            pl.BlockSpec((1, c_in), lambda b, s: (0, 0)),
            pl.BlockSpec((c_in, c_out), lambda b, s: (0, 0)),
            pl.BlockSpec((1, c_out), lambda b, s: (0, 0)),
        ],
        out_specs=[
            pl.BlockSpec((1, Rp, c_out), lambda b, s: (b, s, 0)),
            pl.BlockSpec((8, c_out), lambda b, s: (0, 0)),
        ],
        out_shape=[
            jax.ShapeDtypeStruct((Bb, P, c_out), F32),
            jax.ShapeDtypeStruct((8, c_out), F32),
        ],
    )(x, st, gam, bet, wt, bvec)


def _pool_body(x_ref, st_ref, gam_ref, bet_ref, out_ref, *, count):
    st = st_ref[...]
    mu = st[0:1, :] / count
    var = st[1:2, :] / count - mu * mu
    inv = lax.rsqrt(var + 1e-5)
    x = x_ref[0]
    xh = jnp.maximum((x - mu[None]) * inv[None] * gam_ref[...][None]
                     + bet_ref[...][None], 0.0)
    out_ref[0] = jnp.max(xh, axis=1)


def _pool(x, st, gam, bet, count, R):
    Bb, S, ns, c = x.shape
    grid = (Bb, S // R)
    return pl.pallas_call(
        partial(_pool_body, count=count),
        grid=grid,
        in_specs=[
            pl.BlockSpec((1, R, ns, c), lambda b, s: (b, s, 0, 0)),
            pl.BlockSpec((8, c), lambda b, s: (0, 0)),
            pl.BlockSpec((1, c), lambda b, s: (0, 0)),
            pl.BlockSpec((1, c), lambda b, s: (0, 0)),
        ],
        out_specs=pl.BlockSpec((1, R, c), lambda b, s: (b, s, 0)),
        out_shape=jax.ShapeDtypeStruct((Bb, S, c), F32),
    )(x, st, gam, bet)


def _fp_body(q_ref, srcT_ref, feat_ref, stf_ref, gamf_ref, betf_ref,
             p1_ref, wt_ref, b_ref, y_ref, st_ref, *, count_f, R, n_src, c_f):
    first = (pl.program_id(0) == 0) & (pl.program_id(1) == 0)

    @pl.when(first)
    def _():
        st_ref[...] = jnp.zeros_like(st_ref)

    stf = stf_ref[...]
    muf = stf[0:1, :] / count_f
    varf = stf[1:2, :] / count_f - muf * muf
    invf = lax.rsqrt(varf + 1e-5)
    gamf = gamf_ref[...]
    betf = betf_ref[...]

    q = q_ref[0]
    sx = srcT_ref[0, 0:1, :]
    sy = srcT_ref[0, 1:2, :]
    sz = srcT_ref[0, 2:3, :]
    qx, qy, qz = q[:, 0:1], q[:, 1:2], q[:, 2:3]
    qq = qx * qx + qy * qy + qz * qz
    ss = sx * sx + sy * sy + sz * sz
    cross = jnp.dot(q, srcT_ref[0], preferred_element_type=F32)
    d = (qq + ss) - 2.0 * cross
    iota_s = _bi(jnp.int32, (R, n_src), 1)
    feat = feat_ref[0]

    work = d
    num = jnp.zeros((R, c_f), F32)
    den = jnp.zeros((R, 1), F32)
    for _k in range(3):
        m = jnp.min(work, axis=1, keepdims=True)
        cand = jnp.where(work == m, iota_s, n_src)
        idx = jnp.min(cand, axis=1, keepdims=True)
        ohb = iota_s == idx
        oh = ohb.astype(F32)
        work = jnp.where(ohb, 1e30, work)
        gk = jnp.dot(oh, feat, preferred_element_type=F32, precision=lax.Precision.HIGHEST)
        gk = jnp.maximum((gk - muf) * invf * gamf + betf, 0.0)
        dist = jnp.sqrt(jnp.maximum(m, 0.0) + 1e-8)
        wk = 1.0 / (dist + 1e-8)
        num = num + wk * gk
        den = den + wk
    interp = num / den
    x = jnp.concatenate([p1_ref[0], interp], axis=1)
    yv = jnp.dot(x, wt_ref[...], preferred_element_type=F32) + b_ref[...]
    y_ref[0] = yv
    st_ref[0:1, :] += jnp.sum(yv, axis=0, keepdims=True)
    st_ref[1:2, :] += jnp.sum(yv * yv, axis=0, keepdims=True)


def _fp_interp_conv(q, srcT, feat, stf, gamf, betf, p1, wt, bvec, count_f, R):
    Bb, S, _ = q.shape
    n_src = feat.shape[1]
    c_f = feat.shape[2]
    c_p = p1.shape[2]
    c_cat = c_p + c_f
    c_out = wt.shape[1]
    grid = (Bb, S // R)
    return pl.pallas_call(
        partial(_fp_body, count_f=count_f, R=R, n_src=n_src, c_f=c_f),
        grid=grid,
        in_specs=[
            pl.BlockSpec((1, R, 3), lambda b, s: (b, s, 0)),
            pl.BlockSpec((1, 3, n_src), lambda b, s: (b, 0, 0)),
            pl.BlockSpec((1, n_src, c_f), lambda b, s: (b, 0, 0)),
            pl.BlockSpec((8, c_f), lambda b, s: (0, 0)),
            pl.BlockSpec((1, c_f), lambda b, s: (0, 0)),
            pl.BlockSpec((1, c_f), lambda b, s: (0, 0)),
            pl.BlockSpec((1, R, c_p), lambda b, s: (b, s, 0)),
            pl.BlockSpec((c_cat, c_out), lambda b, s: (0, 0)),
            pl.BlockSpec((1, c_out), lambda b, s: (0, 0)),
        ],
        out_specs=[
            pl.BlockSpec((1, R, c_out), lambda b, s: (b, s, 0)),
            pl.BlockSpec((8, c_out), lambda b, s: (0, 0)),
        ],
        out_shape=[
            jax.ShapeDtypeStruct((Bb, S, c_out), F32),
            jax.ShapeDtypeStruct((8, c_out), F32),
        ],
    )(q, srcT, feat, stf, gamf, betf, p1, wt, bvec)


def _final_body(x_ref, st_ref, gam_ref, bet_ref, wt_ref, b_ref, out_ref, *,
                count):
    st = st_ref[...]
    mu = st[0:1, :] / count
    var = st[1:2, :] / count - mu * mu
    inv = lax.rsqrt(var + 1e-5)
    x = x_ref[0]
    xh = jnp.maximum((x - mu) * inv * gam_ref[...] + bet_ref[...], 0.0)
    logits = jnp.dot(xh, wt_ref[...], preferred_element_type=F32) + b_ref[...]
    m = jnp.max(logits, axis=1, keepdims=True)
    e = jnp.exp(logits - m)
    s = jnp.sum(e, axis=1, keepdims=True)
    out_ref[0] = logits - m - jnp.log(s)


def _final(x, st, gam, bet, wt, bvec, count, Rp):
    Bb, P, c_in = x.shape
    c_out = wt.shape[1]
    grid = (Bb, P // Rp)
    return pl.pallas_call(
        partial(_final_body, count=count),
        grid=grid,
        in_specs=[
            pl.BlockSpec((1, Rp, c_in), lambda b, s: (b, s, 0)),
            pl.BlockSpec((8, c_in), lambda b, s: (0, 0)),
            pl.BlockSpec((1, c_in), lambda b, s: (0, 0)),
            pl.BlockSpec((1, c_in), lambda b, s: (0, 0)),
            pl.BlockSpec((c_in, c_out), lambda b, s: (0, 0)),
            pl.BlockSpec((1, c_out), lambda b, s: (0, 0)),
        ],
        out_specs=pl.BlockSpec((1, Rp, c_out), lambda b, s: (b, s, 0)),
        out_shape=jax.ShapeDtypeStruct((Bb, P, c_out), F32),
    )(x, st, gam, bet, wt, bvec)


def _lyr(params, name, i):
    return (params[name + '_%d_W' % i].T,
            params[name + '_%d_b' % i][None, :],
            params[name + '_%d_g' % i][None, :],
            params[name + '_%d_be' % i][None, :])


@jax.jit
def kernel(xyz, params):
    Bb, n0, _ = xyz.shape
    ns = 32
    xyzT = jnp.transpose(xyz, (0, 2, 1))

    nx1T = _fps(xyzT, 1024)
    nx1 = jnp.transpose(nx1T, (0, 2, 1))
    w1, b1, g1, be1 = _lyr(params, 'sa1', 0)
    w2, b2, g2, be2 = _lyr(params, 'sa1', 1)
    w3, b3, g3, be3 = _lyr(params, 'sa1', 2)
    cnt1 = Bb * 1024 * ns
    y11, st11 = _sa_group_conv(nx1, xyzT, xyz, w1, b1, 0.1, ns, R=8)
    y12, st12 = _mid_layer(y11, st11, g1, be1, w2, b2, cnt1, Rp=512)
    y13, st13 = _mid_layer(y12, st12, g2, be2, w3, b3, cnt1, Rp=512)
    p1 = _pool(y13.reshape(Bb, 1024, ns, -1), st13, g3, be3, cnt1, R=64)

    nx2T = _fps(nx1T, 256)
    nx2 = jnp.transpose(nx2T, (0, 2, 1))
    src2 = jnp.concatenate([nx1, p1], axis=2)
    w1, b1, g1, be1 = _lyr(params, 'sa2', 0)
    w2, b2, g2, be2 = _lyr(params, 'sa2', 1)
    w3, b3, g3, be3 = _lyr(params, 'sa2', 2)
    cnt2 = Bb * 256 * ns
    y21, st21 = _sa_group_conv(nx2, nx1T, src2, w1, b1, 0.2, ns, R=8)
    y22, st22 = _mid_layer(y21, st21, g1, be1, w2, b2, cnt2, Rp=512)
    y23, st23 = _mid_layer(y22, st22, g2, be2, w3, b3, cnt2, Rp=512)
    p2 = _pool(y23.reshape(Bb, 256, ns, -1), st23, g3, be3, cnt2, R=32)

    w1, b1, g1, be1 = _lyr(params, 'fp1', 0)
    w2, b2, g2, be2 = _lyr(params, 'fp1', 1)
    c2 = p2.shape[2]
    id_st = jnp.concatenate([jnp.zeros((1, c2), F32),
                             jnp.full((1, c2), 1.0, F32),
                             jnp.zeros((6, c2), F32)], axis=0)
    id_gam = jnp.full((1, c2), jnp.sqrt(jnp.float32(1.0 + 1e-5)), F32)
    id_bet = jnp.zeros((1, c2), F32)
    yf11, stf11 = _fp_interp_conv(nx1, nx2T, p2, id_st, id_gam, id_bet,
                                  p1, w1, b1, 1, R=32)
    cntf1 = Bb * 1024
    yf12, stf12 = _mid_layer(yf11, stf11, g1, be1, w2, b2, cntf1, Rp=512)

    w1, b1, g1b, be1b = _lyr(params, 'fp0', 0)
    w2, b2, g2b, be2b = _lyr(params, 'fp0', 1)
    yf01, stf01 = _fp_interp_conv(xyz, nx1T, yf12, stf12, g2, be2,
                                  xyz, w1, b1, cntf1, R=16)
    cnt0 = Bb * n0
    yf02, stf02 = _mid_layer(yf01, stf01, g1b, be1b, w2, b2, cnt0, Rp=512)

    wc, bc, gc, bec = _lyr(params, 'cls0', 0)
    yc, stc = _mid_layer(yf02, stf02, g2b, be2b, wc, bc, cnt0, Rp=512)
    wf = params['cls1_W'].T
    bf = params['cls1_b'][None, :]
    out = _final(yc, stc, gc, bec, wf, bf, cnt0, Rp=512)
    return out


# speedup vs baseline: 51.5965x; 1.4096x over previous
import functools
from functools import partial

import jax
import jax.numpy as jnp
from jax import lax
from jax.experimental import pallas as pl
from jax.experimental.pallas import tpu as pltpu

F32 = jnp.float32


def _bi(dtype, shape, dim):
    return lax.broadcasted_iota(dtype, shape, dim)


def _fps_body(xyzT_ref, out_ref, *, npoint):
    Bb = xyzT_ref.shape[0]
    n = xyzT_ref.shape[2]
    x = xyzT_ref[:, 0, :]
    y = xyzT_ref[:, 1, :]
    z = xyzT_ref[:, 2, :]
    iota_n = _bi(jnp.int32, (Bb, n), 1)
    iota_p = _bi(jnp.int32, (Bb, npoint), 1)

    def body(i, st):
        dists, far, ox, oy, oz = st
        oh = (iota_n == far).astype(F32)
        cx = jnp.sum(x * oh, axis=1, keepdims=True)
        cy = jnp.sum(y * oh, axis=1, keepdims=True)
        cz = jnp.sum(z * oh, axis=1, keepdims=True)
        d = (x - cx) ** 2 + (y - cy) ** 2 + (z - cz) ** 2
        dists = jnp.minimum(dists, d)
        m = jnp.max(dists, axis=1, keepdims=True)
        cand = jnp.where(dists == m, iota_n, n)
        far = jnp.min(cand, axis=1, keepdims=True)
        slot = iota_p == i
        ox = jnp.where(slot, cx, ox)
        oy = jnp.where(slot, cy, oy)
        oz = jnp.where(slot, cz, oz)
        return dists, far, ox, oy, oz

    dists0 = jnp.full((Bb, n), 1e10, F32)
    far0 = jnp.zeros((Bb, 1), jnp.int32)
    zp = jnp.zeros((Bb, npoint), F32)
    _, _, ox, oy, oz = lax.fori_loop(0, npoint, body, (dists0, far0, zp, zp, zp))
    out_ref[:, 0, :] = ox
    out_ref[:, 1, :] = oy
    out_ref[:, 2, :] = oz


def _fps(xyzT, npoint):
    Bb, _, n = xyzT.shape
    return pl.pallas_call(
        partial(_fps_body, npoint=npoint),
        out_shape=jax.ShapeDtypeStruct((Bb, 3, npoint), F32),
    )(xyzT)


def _sa_body(q_ref, srcT_ref, src_ref, wt_ref, b_ref, y_ref, st_ref, *,
             radius, ns, R, n_src, c_in):
    first = (pl.program_id(0) == 0) & (pl.program_id(1) == 0)

    @pl.when(first)
    def _():
        st_ref[...] = jnp.zeros_like(st_ref)

    q = q_ref[0]
    sx = srcT_ref[0, 0:1, :]
    sy = srcT_ref[0, 1:2, :]
    sz = srcT_ref[0, 2:3, :]
    qx, qy, qz = q[:, 0:1], q[:, 1:2], q[:, 2:3]
    qq = qx * qx + qy * qy + qz * qz
    ss = sx * sx + sy * sy + sz * sz
    cross = jnp.dot(q, srcT_ref[0], preferred_element_type=F32)
    d = (qq + ss) - 2.0 * cross
    mask = jnp.logical_not(d > radius * radius)
    maskf = mask.astype(F32)

    nch = n_src // 128
    tri128 = (_bi(jnp.int32, (128, 128), 0)
              <= _bi(jnp.int32, (128, 128), 1)).astype(F32)
    tric = (_bi(jnp.int32, (nch, nch), 0)
            < _bi(jnp.int32, (nch, nch), 1)).astype(F32)
    m3 = maskf.reshape(R, nch, 128)
    within = jnp.dot(m3.reshape(R * nch, 128), tri128,
                     preferred_element_type=F32).reshape(R, nch, 128)
    tot = jnp.sum(m3, axis=2)
    carry = jnp.dot(tot, tric, preferred_element_type=F32)
    pos = (within + carry[:, :, None]).reshape(R, n_src)

    jp1 = (_bi(jnp.int32, (1, ns, 1), 1) + 1).astype(F32)
    sel = ((pos[:, None, :] == jp1) & mask[:, None, :]).astype(F32)
    g = jnp.dot(sel.reshape(R * ns, n_src), src_ref[0],
                preferred_element_type=F32,
                precision=lax.Precision.HIGHEST).reshape(R, ns, c_in)

    cnt = jnp.sum(maskf, axis=1).reshape(R, 1, 1)
    g0 = g[:, 0:1, :]
    src0 = src_ref[0][0:1, :].reshape(1, 1, c_in)
    base = jnp.where(cnt > 0, g0, src0)
    jf = _bi(jnp.int32, (1, ns, 1), 1).astype(F32)
    g = jnp.where(jf < cnt, g, base)

    gx = g[:, :, 0:3] - q[:, None, :]
    if c_in > 3:
        g = jnp.concatenate([gx, g[:, :, 3:]], axis=2)
    else:
        g = gx

    x2 = g.reshape(R * ns, c_in)
    yv = jnp.dot(x2, wt_ref[...], preferred_element_type=F32) + b_ref[...]
    y_ref[0] = yv
    s1 = jnp.sum(yv, axis=0, keepdims=True)
    s2 = jnp.sum(yv * yv, axis=0, keepdims=True)
    st_ref[0:1, :] += s1
    st_ref[1:2, :] += s2


def _sa_group_conv(q, srcT, src, wt, bvec, radius, ns, R):
    Bb, S, _ = q.shape
    n_src = src.shape[1]
    c_in = src.shape[2]
    c_out = wt.shape[1]
    grid = (Bb, S // R)
    return pl.pallas_call(
        partial(_sa_body, radius=radius, ns=ns, R=R, n_src=n_src, c_in=c_in),
        grid=grid,
        in_specs=[
            pl.BlockSpec((1, R, 3), lambda b, s: (b, s, 0)),
            pl.BlockSpec((1, 3, n_src), lambda b, s: (b, 0, 0)),
            pl.BlockSpec((1, n_src, c_in), lambda b, s: (b, 0, 0)),
            pl.BlockSpec((c_in, c_out), lambda b, s: (0, 0)),
            pl.BlockSpec((1, c_out), lambda b, s: (0, 0)),
        ],
        out_specs=[
            pl.BlockSpec((1, R * ns, c_out), lambda b, s: (b, s, 0)),
            pl.BlockSpec((8, c_out), lambda b, s: (0, 0)),
        ],
        out_shape=[
            jax.ShapeDtypeStruct((Bb, S * ns, c_out), F32),
            jax.ShapeDtypeStruct((8, c_out), F32),
        ],
    )(q, srcT, src, wt, bvec)


def _mid_body(x_ref, st_ref, gam_ref, bet_ref, wt_ref, b_ref,
              y_ref, sto_ref, *, count):
    first = (pl.program_id(0) == 0) & (pl.program_id(1) == 0)

    @pl.when(first)
    def _():
        sto_ref[...] = jnp.zeros_like(sto_ref)

    st = st_ref[...]
    mu = st[0:1, :] / count
    var = st[1:2, :] / count - mu * mu
    inv = lax.rsqrt(var + 1e-5)
    x = x_ref[0]
    xh = jnp.maximum((x - mu) * inv * gam_ref[...] + bet_ref[...], 0.0)
    yv = jnp.dot(xh, wt_ref[...], preferred_element_type=F32) + b_ref[...]
    y_ref[0] = yv
    sto_ref[0:1, :] += jnp.sum(yv, axis=0, keepdims=True)
    sto_ref[1:2, :] += jnp.sum(yv * yv, axis=0, keepdims=True)


def _mid_layer(x, st, gam, bet, wt, bvec, count, Rp):
    Bb, P, c_in = x.shape
    c_out = wt.shape[1]
    grid = (Bb, P // Rp)
    return pl.pallas_call(
        partial(_mid_body, count=count),
        grid=grid,
        in_specs=[
            pl.BlockSpec((1, Rp, c_in), lambda b, s: (b, s, 0)),
            pl.BlockSpec((8, c_in), lambda b, s: (0, 0)),
            pl.BlockSpec((1, c_in), lambda b, s: (0, 0)),
            pl.BlockSpec((1, c_in), lambda b, s: (0, 0)),
            pl.BlockSpec((c_in, c_out), lambda b, s: (0, 0)),
            pl.BlockSpec((1, c_out), lambda b, s: (0, 0)),
        ],
        out_specs=[
            pl.BlockSpec((1, Rp, c_out), lambda b, s: (b, s, 0)),
            pl.BlockSpec((8, c_out), lambda b, s: (0, 0)),
        ],
        out_shape=[
            jax.ShapeDtypeStruct((Bb, P, c_out), F32),
            jax.ShapeDtypeStruct((8, c_out), F32),
        ],
    )(x, st, gam, bet, wt, bvec)


def _pool_body(x_ref, st_ref, gam_ref, bet_ref, out_ref, *, count):
    st = st_ref[...]
    mu = st[0:1, :] / count
    var = st[1:2, :] / count - mu * mu
    inv = lax.rsqrt(var + 1e-5)
    x = x_ref[0]
    xh = jnp.maximum((x - mu[None]) * inv[None] * gam_ref[...][None]
                     + bet_ref[...][None], 0.0)
    out_ref[0] = jnp.max(xh, axis=1)


def _pool(x, st, gam, bet, count, R):
    Bb, S, ns, c = x.shape
    grid = (Bb, S // R)
    return pl.pallas_call(
        partial(_pool_body, count=count),
        grid=grid,
        in_specs=[
            pl.BlockSpec((1, R, ns, c), lambda b, s: (b, s, 0, 0)),
            pl.BlockSpec((8, c), lambda b, s: (0, 0)),
            pl.BlockSpec((1, c), lambda b, s: (0, 0)),
            pl.BlockSpec((1, c), lambda b, s: (0, 0)),
        ],
        out_specs=pl.BlockSpec((1, R, c), lambda b, s: (b, s, 0)),
        out_shape=jax.ShapeDtypeStruct((Bb, S, c), F32),
    )(x, st, gam, bet)


def _fp_body(q_ref, srcT_ref, feat_ref, stf_ref, gamf_ref, betf_ref,
             p1_ref, wt_ref, b_ref, y_ref, st_ref, *, count_f, R, n_src, c_f):
    first = (pl.program_id(0) == 0) & (pl.program_id(1) == 0)

    @pl.when(first)
    def _():
        st_ref[...] = jnp.zeros_like(st_ref)

    stf = stf_ref[...]
    muf = stf[0:1, :] / count_f
    varf = stf[1:2, :] / count_f - muf * muf
    invf = lax.rsqrt(varf + 1e-5)
    gamf = gamf_ref[...]
    betf = betf_ref[...]

    q = q_ref[0]
    sx = srcT_ref[0, 0:1, :]
    sy = srcT_ref[0, 1:2, :]
    sz = srcT_ref[0, 2:3, :]
    qx, qy, qz = q[:, 0:1], q[:, 1:2], q[:, 2:3]
    qq = qx * qx + qy * qy + qz * qz
    ss = sx * sx + sy * sy + sz * sz
    cross = jnp.dot(q, srcT_ref[0], preferred_element_type=F32)
    d = (qq + ss) - 2.0 * cross
    iota_s = _bi(jnp.int32, (R, n_src), 1)
    feat = feat_ref[0]

    work = d
    num = jnp.zeros((R, c_f), F32)
    den = jnp.zeros((R, 1), F32)
    for _k in range(3):
        m = jnp.min(work, axis=1, keepdims=True)
        cand = jnp.where(work == m, iota_s, n_src)
        idx = jnp.min(cand, axis=1, keepdims=True)
        ohb = iota_s == idx
        oh = ohb.astype(F32)
        work = jnp.where(ohb, 1e30, work)
        gk = jnp.dot(oh, feat, preferred_element_type=F32, precision=lax.Precision.HIGHEST)
        gk = jnp.maximum((gk - muf) * invf * gamf + betf, 0.0)
        dist = jnp.sqrt(jnp.maximum(m, 0.0) + 1e-8)
        wk = 1.0 / (dist + 1e-8)
        num = num + wk * gk
        den = den + wk
    interp = num / den
    x = jnp.concatenate([p1_ref[0], interp], axis=1)
    yv = jnp.dot(x, wt_ref[...], preferred_element_type=F32) + b_ref[...]
    y_ref[0] = yv
    st_ref[0:1, :] += jnp.sum(yv, axis=0, keepdims=True)
    st_ref[1:2, :] += jnp.sum(yv * yv, axis=0, keepdims=True)


def _fp_interp_conv(q, srcT, feat, stf, gamf, betf, p1, wt, bvec, count_f, R):
    Bb, S, _ = q.shape
    n_src = feat.shape[1]
    c_f = feat.shape[2]
    c_p = p1.shape[2]
    c_cat = c_p + c_f
    c_out = wt.shape[1]
    grid = (Bb, S // R)
    return pl.pallas_call(
        partial(_fp_body, count_f=count_f, R=R, n_src=n_src, c_f=c_f),
        grid=grid,
        in_specs=[
            pl.BlockSpec((1, R, 3), lambda b, s: (b, s, 0)),
            pl.BlockSpec((1, 3, n_src), lambda b, s: (b, 0, 0)),
            pl.BlockSpec((1, n_src, c_f), lambda b, s: (b, 0, 0)),
            pl.BlockSpec((8, c_f), lambda b, s: (0, 0)),
            pl.BlockSpec((1, c_f), lambda b, s: (0, 0)),
            pl.BlockSpec((1, c_f), lambda b, s: (0, 0)),
            pl.BlockSpec((1, R, c_p), lambda b, s: (b, s, 0)),
            pl.BlockSpec((c_cat, c_out), lambda b, s: (0, 0)),
            pl.BlockSpec((1, c_out), lambda b, s: (0, 0)),
        ],
        out_specs=[
            pl.BlockSpec((1, R, c_out), lambda b, s: (b, s, 0)),
            pl.BlockSpec((8, c_out), lambda b, s: (0, 0)),
        ],
        out_shape=[
            jax.ShapeDtypeStruct((Bb, S, c_out), F32),
            jax.ShapeDtypeStruct((8, c_out), F32),
        ],
    )(q, srcT, feat, stf, gamf, betf, p1, wt, bvec)


def _final_body(x_ref, st_ref, gam_ref, bet_ref, wt_ref, b_ref, out_ref, *,
                count):
    st = st_ref[...]
    mu = st[0:1, :] / count
    var = st[1:2, :] / count - mu * mu
    inv = lax.rsqrt(var + 1e-5)
    x = x_ref[0]
    xh = jnp.maximum((x - mu) * inv * gam_ref[...] + bet_ref[...], 0.0)
    logits = jnp.dot(xh, wt_ref[...], preferred_element_type=F32) + b_ref[...]
    m = jnp.max(logits, axis=1, keepdims=True)
    e = jnp.exp(logits - m)
    s = jnp.sum(e, axis=1, keepdims=True)
    out_ref[0] = logits - m - jnp.log(s)


def _final(x, st, gam, bet, wt, bvec, count, Rp):
    Bb, P, c_in = x.shape
    c_out = wt.shape[1]
    grid = (Bb, P // Rp)
    return pl.pallas_call(
        partial(_final_body, count=count),
        grid=grid,
        in_specs=[
            pl.BlockSpec((1, Rp, c_in), lambda b, s: (b, s, 0)),
            pl.BlockSpec((8, c_in), lambda b, s: (0, 0)),
            pl.BlockSpec((1, c_in), lambda b, s: (0, 0)),
            pl.BlockSpec((1, c_in), lambda b, s: (0, 0)),
            pl.BlockSpec((c_in, c_out), lambda b, s: (0, 0)),
            pl.BlockSpec((1, c_out), lambda b, s: (0, 0)),
        ],
        out_specs=pl.BlockSpec((1, Rp, c_out), lambda b, s: (b, s, 0)),
        out_shape=jax.ShapeDtypeStruct((Bb, P, c_out), F32),
    )(x, st, gam, bet, wt, bvec)


def _lyr(params, name, i):
    return (params[name + '_%d_W' % i].T,
            params[name + '_%d_b' % i][None, :],
            params[name + '_%d_g' % i][None, :],
            params[name + '_%d_be' % i][None, :])


@jax.jit
def kernel(xyz, params):
    Bb, n0, _ = xyz.shape
    ns = 32
    xyzT = jnp.transpose(xyz, (0, 2, 1))

    nx1T = _fps(xyzT, 1024)
    nx1 = jnp.transpose(nx1T, (0, 2, 1))
    w1, b1, g1, be1 = _lyr(params, 'sa1', 0)
    w2, b2, g2, be2 = _lyr(params, 'sa1', 1)
    w3, b3, g3, be3 = _lyr(params, 'sa1', 2)
    cnt1 = Bb * 1024 * ns
    y11, st11 = _sa_group_conv(nx1, xyzT, xyz, w1, b1, 0.1, ns, R=16)
    y12, st12 = _mid_layer(y11, st11, g1, be1, w2, b2, cnt1, Rp=1024)
    y13, st13 = _mid_layer(y12, st12, g2, be2, w3, b3, cnt1, Rp=1024)
    p1 = _pool(y13.reshape(Bb, 1024, ns, -1), st13, g3, be3, cnt1, R=64)

    nx2T = _fps(nx1T, 256)
    nx2 = jnp.transpose(nx2T, (0, 2, 1))
    src2 = jnp.concatenate([nx1, p1], axis=2)
    w1, b1, g1, be1 = _lyr(params, 'sa2', 0)
    w2, b2, g2, be2 = _lyr(params, 'sa2', 1)
    w3, b3, g3, be3 = _lyr(params, 'sa2', 2)
    cnt2 = Bb * 256 * ns
    y21, st21 = _sa_group_conv(nx2, nx1T, src2, w1, b1, 0.2, ns, R=32)
    y22, st22 = _mid_layer(y21, st21, g1, be1, w2, b2, cnt2, Rp=1024)
    y23, st23 = _mid_layer(y22, st22, g2, be2, w3, b3, cnt2, Rp=1024)
    p2 = _pool(y23.reshape(Bb, 256, ns, -1), st23, g3, be3, cnt2, R=64)

    w1, b1, g1, be1 = _lyr(params, 'fp1', 0)
    w2, b2, g2, be2 = _lyr(params, 'fp1', 1)
    c2 = p2.shape[2]
    id_st = jnp.concatenate([jnp.zeros((1, c2), F32),
                             jnp.full((1, c2), 1.0, F32),
                             jnp.zeros((6, c2), F32)], axis=0)
    id_gam = jnp.full((1, c2), jnp.sqrt(jnp.float32(1.0 + 1e-5)), F32)
    id_bet = jnp.zeros((1, c2), F32)
    yf11, stf11 = _fp_interp_conv(nx1, nx2T, p2, id_st, id_gam, id_bet,
                                  p1, w1, b1, 1, R=64)
    cntf1 = Bb * 1024
    yf12, stf12 = _mid_layer(yf11, stf11, g1, be1, w2, b2, cntf1, Rp=1024)

    w1, b1, g1b, be1b = _lyr(params, 'fp0', 0)
    w2, b2, g2b, be2b = _lyr(params, 'fp0', 1)
    yf01, stf01 = _fp_interp_conv(xyz, nx1T, yf12, stf12, g2, be2,
                                  xyz, w1, b1, cntf1, R=64)
    cnt0 = Bb * n0
    yf02, stf02 = _mid_layer(yf01, stf01, g1b, be1b, w2, b2, cnt0, Rp=1024)

    wc, bc, gc, bec = _lyr(params, 'cls0', 0)
    yc, stc = _mid_layer(yf02, stf02, g2b, be2b, wc, bc, cnt0, Rp=1024)
    wf = params['cls1_W'].T
    bf = params['cls1_b'][None, :]
    out = _final(yc, stc, gc, bec, wf, bf, cnt0, Rp=1024)
    return out
